# Initial kernel scaffold; baseline (speedup 1.0000x reference)
import jax
import jax.numpy as jnp
from jax.experimental import pallas as pl


def kernel(input_ids, topk_id, topk_prob, corpus, sw_mask, bd_mask):
    raise NotImplementedError("write your pallas kernel here")


# speedup vs baseline: 11.1343x; 11.1343x over previous
import functools

import jax
import jax.numpy as jnp
from jax import lax
from jax.experimental import pallas as pl
from jax.experimental.pallas import tpu as pltpu
from jax.experimental.pallas import tpu_sc as plsc

_N = 3
_BETA = 0.9
_SW_COEFF = 1.0
_VOCAB = 100000
_L = 32768
_K = 64

_NT = 16
_CHUNK = _L // _NT
_ZSLICE = 18752
_ZTOT = _NT * _ZSLICE
_TOT_OFF = _ZTOT
_ZSIZE = _ZTOT + _NT * 32
_PCHUNK = 6256
_PLAST = _VOCAB - 15 * _PCHUNK
_T1_SLOT = 300016
_T2_SLOT = 300017


def _body(corpus_ref, lasts_ref, topk_ref, cmask_ref, out_ref,
          ztab, cv, zbuf, lasts_v, topk_v, cmask_v,
          ones_s, val1_s, val2_s, idx0_s, idx1_s, idx2_s,
          tidx1, tidx2, tval1, tval2, gidx, gval, finalbuf):
  sid = lax.axis_index("s")
  zero16 = jnp.zeros((16,), jnp.float32)
  ones16 = jnp.ones((16,), jnp.float32)
  lanes = lax.iota(jnp.int32, 16)

  def _zfill(i, _):
    zbuf[pl.ds(i * 16, 16)] = zero16
    return 0
  lax.fori_loop(0, _PCHUNK // 16, _zfill, 0)
  for t in range(8):
    ones_s[pl.ds(t * 16, 16)] = ones16

  @pl.when(sid < _NT - 1)
  def _():
    pltpu.sync_copy(zbuf, out_ref.at[pl.ds(sid * _PCHUNK, _PCHUNK)])

  @pl.when(sid == _NT - 1)
  def _():
    pltpu.sync_copy(zbuf.at[pl.ds(0, _PLAST)],
                    out_ref.at[pl.ds((_NT - 1) * _PCHUNK, _PLAST)])

  zb = sid * _ZSLICE
  pltpu.sync_copy(zbuf, ztab.at[pl.ds(zb, _PCHUNK)])
  pltpu.sync_copy(zbuf, ztab.at[pl.ds(zb + _PCHUNK, _PCHUNK)])
  rem = _ZSLICE - 2 * _PCHUNK
  pltpu.sync_copy(zbuf.at[pl.ds(0, rem)],
                  ztab.at[pl.ds(zb + 2 * _PCHUNK, rem)])

  pltpu.sync_copy(corpus_ref.at[pl.ds(sid * _CHUNK, _CHUNK + 16)], cv)
  pltpu.sync_copy(lasts_ref, lasts_v)
  last2v = lasts_v[pl.ds(0, 16)]
  lastv = lasts_v[pl.ds(16, 16)]

  plsc.subcore_barrier()

  def _group(g, carry):
    t1acc, t2acc = carry
    for t in range(8):
      p = g * 128 + t * 16
      cv0 = cv[pl.ds(p, 16)]
      cv1 = cv[pl.ds(p + 1, 16)]
      cv2 = cv[pl.ds(p + 2, 16)]
      posv = sid * _CHUNK + p + lanes
      m1 = (cv0 == lastv) & (posv <= _L - 2)
      m2 = (cv0 == last2v) & (cv1 == lastv) & (posv <= _L - 3)
      m1f = jnp.where(m1, 1.0, 0.0).astype(jnp.float32)
      m2f = jnp.where(m2, 1.0, 0.0).astype(jnp.float32)
      t1acc = t1acc + m1f
      t2acc = t2acc + m2f
      idx0_s[pl.ds(t * 16, 16)] = cv0
      idx1_s[pl.ds(t * 16, 16)] = cv1 + _VOCAB
      idx2_s[pl.ds(t * 16, 16)] = cv2 + 2 * _VOCAB
      val1_s[pl.ds(t * 16, 16)] = m1f
      val2_s[pl.ds(t * 16, 16)] = m2f
    pltpu.sync_copy(ones_s, ztab.at[idx0_s], add=True)
    pltpu.sync_copy(val1_s, ztab.at[idx1_s], add=True)
    pltpu.sync_copy(val2_s, ztab.at[idx2_s], add=True)
    return t1acc, t2acc

  t1acc, t2acc = lax.fori_loop(
      0, _CHUNK // 128, _group, (zero16, zero16))

  tidx1[pl.ds(0, 16)] = jnp.full((16,), _T1_SLOT, jnp.int32)
  tidx2[pl.ds(0, 16)] = jnp.full((16,), _T2_SLOT, jnp.int32)
  tval1[pl.ds(0, 16)] = t1acc
  tval2[pl.ds(0, 16)] = t2acc
  pltpu.sync_copy(tval1, ztab.at[tidx1], add=True)
  pltpu.sync_copy(tval2, ztab.at[tidx2], add=True)

  plsc.subcore_barrier()

  @pl.when(sid == 0)
  def _():
    pltpu.sync_copy(topk_ref, topk_v)
    pltpu.sync_copy(cmask_ref, cmask_v)
    zero16i = jnp.zeros((16,), jnp.int32)
    for g in range(4):
      tk = topk_v[pl.ds(g * 16, 16)]
      gidx[pl.ds(g * 16, 16)] = tk
      gidx[pl.ds(64 + g * 16, 16)] = tk + _VOCAB
      gidx[pl.ds(128 + g * 16, 16)] = tk + 2 * _VOCAB
    gidx[pl.ds(192, 16)] = jnp.full((16,), _T1_SLOT, jnp.int32)
    gidx[pl.ds(208, 16)] = jnp.full((16,), _T2_SLOT, jnp.int32)
    gidx[pl.ds(224, 16)] = zero16i
    gidx[pl.ds(240, 16)] = zero16i
    pltpu.sync_copy(ztab.at[gidx.at[pl.ds(0, 128)]], gval.at[pl.ds(0, 128)])
    pltpu.sync_copy(ztab.at[gidx.at[pl.ds(128, 128)]],
                    gval.at[pl.ds(128, 128)])
    t1 = gval[pl.ds(192, 16)]
    t2 = gval[pl.ds(208, 16)]

    for g in range(4):
      c0 = gval[pl.ds(g * 16, 16)]
      c1 = gval[pl.ds(64 + g * 16, 16)]
      c2 = gval[pl.ds(128 + g * 16, 16)]
      bd = cmask_v[pl.ds(g * 16, 16)]
      sw = cmask_v[pl.ds(64 + g * 16, 16)]
      notbd = bd == 0.0
      remaining = ones16
      hit2 = (c2 > 0.0) & notbd
      score = jnp.where(hit2, _BETA * (c2 / (t2 + 1.0)), 0.0)
      remaining = jnp.where(hit2, remaining * (1.0 - _BETA), remaining)
      hit1 = (c1 > 0.0) & notbd
      score = score + jnp.where(hit1, remaining * _BETA * (c1 / (t1 + 1.0)),
                                0.0)
      remaining = jnp.where(hit1, remaining * (1.0 - _BETA), remaining)
      hit0 = (c0 > 0.0) & notbd
      score = score + jnp.where(hit0, remaining * (c0 / float(_L)), 0.0)
      fin = jnp.where(bd > 0.0, 0.0,
                      jnp.where(sw > 0.0, _SW_COEFF * score, score))
      finalbuf[pl.ds(g * 16, 16)] = fin

    pltpu.sync_copy(finalbuf, out_ref.at[topk_v])


@functools.partial(
    pl.kernel,
    out_type=jax.ShapeDtypeStruct((_VOCAB,), jnp.float32),
    mesh=plsc.VectorSubcoreMesh(
        core_axis_name="c", subcore_axis_name="s", num_cores=1),
    scratch_types=[
        pltpu.VMEM_SHARED((_ZSIZE,), jnp.float32),
        pltpu.VMEM((_CHUNK + 16,), jnp.int32),
        pltpu.VMEM((_PCHUNK,), jnp.float32),
        pltpu.VMEM((32,), jnp.int32),
        pltpu.VMEM((_K,), jnp.int32),
        pltpu.VMEM((2 * _K,), jnp.float32),
        pltpu.VMEM((128,), jnp.float32),
        pltpu.VMEM((128,), jnp.float32),
        pltpu.VMEM((128,), jnp.float32),
        pltpu.VMEM((128,), jnp.int32),
        pltpu.VMEM((128,), jnp.int32),
        pltpu.VMEM((128,), jnp.int32),
        pltpu.VMEM((16,), jnp.int32),
        pltpu.VMEM((16,), jnp.int32),
        pltpu.VMEM((16,), jnp.float32),
        pltpu.VMEM((16,), jnp.float32),
        pltpu.VMEM((256,), jnp.int32),
        pltpu.VMEM((256,), jnp.float32),
        pltpu.VMEM((_K,), jnp.float32),
    ],
)
def _ngram_sc(corpus_ref, lasts_ref, topk_ref, cmask_ref, out_ref, *scratch):
  _body(corpus_ref, lasts_ref, topk_ref, cmask_ref, out_ref, *scratch)


@jax.jit
def kernel(input_ids, topk_id, topk_prob, corpus, sw_mask, bd_mask):
  del topk_prob
  corpus_pad = jnp.concatenate([corpus, jnp.zeros((16,), jnp.int32)])
  lasts = jnp.concatenate([
      jnp.full((16,), input_ids[-2], jnp.int32),
      jnp.full((16,), input_ids[-1], jnp.int32),
  ])
  cmask = jnp.concatenate([
      bd_mask[topk_id].astype(jnp.float32),
      sw_mask[topk_id].astype(jnp.float32),
  ])
  return _ngram_sc(corpus_pad, lasts, topk_id, cmask)


# speedup vs baseline: 11.7691x; 1.0570x over previous
import functools

import jax
import jax.numpy as jnp
from jax import lax
from jax.experimental import pallas as pl
from jax.experimental.pallas import tpu as pltpu
from jax.experimental.pallas import tpu_sc as plsc

_N = 3
_BETA = 0.9
_SW_COEFF = 1.0
_VOCAB = 100000
_L = 32768
_K = 64

_NT = 16
_CHUNK = _L // _NT
_ZSLICE = 18752
_ZTOT = _NT * _ZSLICE
_TOT_OFF = _ZTOT
_ZSIZE = _ZTOT + _NT * 32
_PCHUNK = 6256
_PLAST = _VOCAB - 15 * _PCHUNK
_T1_SLOT = 300016
_T2_SLOT = 300017


def _body(corpus_ref, lasts_ref, topk_ref, cmask_ref, out_ref,
          ztab, cv, zbuf, lasts_v, topk_v, cmask_v,
          ones_s, val1_s, val2_s, idx0_s, idx1_s, idx2_s,
          tidx1, tidx2, tval1, tval2, gidx, gval, finalbuf):
  sid = lax.axis_index("s")
  zero16 = jnp.zeros((16,), jnp.float32)
  ones16 = jnp.ones((16,), jnp.float32)
  lanes = lax.iota(jnp.int32, 16)

  def _load_phase(s0, s1, s2, s3, s4, s5):
    d_cv = pltpu.async_copy(
        corpus_ref.at[pl.ds(sid * _CHUNK, _CHUNK + 16)], cv, s0)
    d_lasts = pltpu.async_copy(lasts_ref, lasts_v, s1)

    def _zfill(i, _):
      zbuf[pl.ds(i * 16, 16)] = zero16
      return 0
    lax.fori_loop(0, _PCHUNK // 16, _zfill, 0)
    for t in range(8):
      ones_s[pl.ds(t * 16, 16)] = ones16

    pofs = jnp.where(sid == _NT - 1, _VOCAB - _PCHUNK, sid * _PCHUNK)
    zb = sid * _ZSLICE
    rem = _ZSLICE - 2 * _PCHUNK
    d_z = [
        pltpu.async_copy(zbuf, out_ref.at[pl.ds(pofs, _PCHUNK)], s2),
        pltpu.async_copy(zbuf, ztab.at[pl.ds(zb, _PCHUNK)], s3),
        pltpu.async_copy(zbuf, ztab.at[pl.ds(zb + _PCHUNK, _PCHUNK)], s4),
        pltpu.async_copy(zbuf.at[pl.ds(0, rem)],
                         ztab.at[pl.ds(zb + 2 * _PCHUNK, rem)], s5),
    ]
    for d in d_z:
      d.wait()
    d_cv.wait()
    d_lasts.wait()

  pl.run_scoped(
      _load_phase,
      s0=pltpu.SemaphoreType.DMA(()), s1=pltpu.SemaphoreType.DMA(()),
      s2=pltpu.SemaphoreType.DMA(()), s3=pltpu.SemaphoreType.DMA(()),
      s4=pltpu.SemaphoreType.DMA(()), s5=pltpu.SemaphoreType.DMA(()))

  last2v = lasts_v[pl.ds(0, 16)]
  lastv = lasts_v[pl.ds(16, 16)]

  plsc.subcore_barrier()

  def _group(g, carry):
    t1acc, t2acc = carry
    for t in range(8):
      p = g * 128 + t * 16
      cv0 = cv[pl.ds(p, 16)]
      cv1 = cv[pl.ds(p + 1, 16)]
      cv2 = cv[pl.ds(p + 2, 16)]
      posv = sid * _CHUNK + p + lanes
      m1 = (cv0 == lastv) & (posv <= _L - 2)
      m2 = (cv0 == last2v) & (cv1 == lastv) & (posv <= _L - 3)
      m1f = jnp.where(m1, 1.0, 0.0).astype(jnp.float32)
      m2f = jnp.where(m2, 1.0, 0.0).astype(jnp.float32)
      t1acc = t1acc + m1f
      t2acc = t2acc + m2f
      idx0_s[pl.ds(t * 16, 16)] = cv0
      idx1_s[pl.ds(t * 16, 16)] = cv1 + _VOCAB
      idx2_s[pl.ds(t * 16, 16)] = cv2 + 2 * _VOCAB
      val1_s[pl.ds(t * 16, 16)] = m1f
      val2_s[pl.ds(t * 16, 16)] = m2f
    pltpu.sync_copy(ones_s, ztab.at[idx0_s], add=True)
    pltpu.sync_copy(val1_s, ztab.at[idx1_s], add=True)
    pltpu.sync_copy(val2_s, ztab.at[idx2_s], add=True)
    return t1acc, t2acc

  t1acc, t2acc = lax.fori_loop(
      0, _CHUNK // 128, _group, (zero16, zero16))

  tidx1[pl.ds(0, 16)] = jnp.full((16,), _T1_SLOT, jnp.int32)
  tidx2[pl.ds(0, 16)] = jnp.full((16,), _T2_SLOT, jnp.int32)
  tval1[pl.ds(0, 16)] = t1acc
  tval2[pl.ds(0, 16)] = t2acc
  pltpu.sync_copy(tval1, ztab.at[tidx1], add=True)
  pltpu.sync_copy(tval2, ztab.at[tidx2], add=True)

  plsc.subcore_barrier()

  @pl.when(sid == 0)
  def _():
    pltpu.sync_copy(topk_ref, topk_v)
    pltpu.sync_copy(cmask_ref, cmask_v)
    zero16i = jnp.zeros((16,), jnp.int32)
    for g in range(4):
      tk = topk_v[pl.ds(g * 16, 16)]
      gidx[pl.ds(g * 16, 16)] = tk
      gidx[pl.ds(64 + g * 16, 16)] = tk + _VOCAB
      gidx[pl.ds(128 + g * 16, 16)] = tk + 2 * _VOCAB
    gidx[pl.ds(192, 16)] = jnp.full((16,), _T1_SLOT, jnp.int32)
    gidx[pl.ds(208, 16)] = jnp.full((16,), _T2_SLOT, jnp.int32)
    gidx[pl.ds(224, 16)] = zero16i
    gidx[pl.ds(240, 16)] = zero16i
    pltpu.sync_copy(ztab.at[gidx.at[pl.ds(0, 128)]], gval.at[pl.ds(0, 128)])
    pltpu.sync_copy(ztab.at[gidx.at[pl.ds(128, 128)]],
                    gval.at[pl.ds(128, 128)])
    t1 = gval[pl.ds(192, 16)]
    t2 = gval[pl.ds(208, 16)]

    for g in range(4):
      c0 = gval[pl.ds(g * 16, 16)]
      c1 = gval[pl.ds(64 + g * 16, 16)]
      c2 = gval[pl.ds(128 + g * 16, 16)]
      bd = cmask_v[pl.ds(g * 16, 16)]
      sw = cmask_v[pl.ds(64 + g * 16, 16)]
      notbd = bd == 0.0
      remaining = ones16
      hit2 = (c2 > 0.0) & notbd
      score = jnp.where(hit2, _BETA * (c2 / (t2 + 1.0)), 0.0)
      remaining = jnp.where(hit2, remaining * (1.0 - _BETA), remaining)
      hit1 = (c1 > 0.0) & notbd
      score = score + jnp.where(hit1, remaining * _BETA * (c1 / (t1 + 1.0)),
                                0.0)
      remaining = jnp.where(hit1, remaining * (1.0 - _BETA), remaining)
      hit0 = (c0 > 0.0) & notbd
      score = score + jnp.where(hit0, remaining * (c0 / float(_L)), 0.0)
      fin = jnp.where(bd > 0.0, 0.0,
                      jnp.where(sw > 0.0, _SW_COEFF * score, score))
      finalbuf[pl.ds(g * 16, 16)] = fin

    pltpu.sync_copy(finalbuf, out_ref.at[topk_v])


@functools.partial(
    pl.kernel,
    out_type=jax.ShapeDtypeStruct((_VOCAB,), jnp.float32),
    mesh=plsc.VectorSubcoreMesh(
        core_axis_name="c", subcore_axis_name="s", num_cores=1),
    scratch_types=[
        pltpu.VMEM_SHARED((_ZSIZE,), jnp.float32),
        pltpu.VMEM((_CHUNK + 16,), jnp.int32),
        pltpu.VMEM((_PCHUNK,), jnp.float32),
        pltpu.VMEM((32,), jnp.int32),
        pltpu.VMEM((_K,), jnp.int32),
        pltpu.VMEM((2 * _K,), jnp.float32),
        pltpu.VMEM((128,), jnp.float32),
        pltpu.VMEM((128,), jnp.float32),
        pltpu.VMEM((128,), jnp.float32),
        pltpu.VMEM((128,), jnp.int32),
        pltpu.VMEM((128,), jnp.int32),
        pltpu.VMEM((128,), jnp.int32),
        pltpu.VMEM((16,), jnp.int32),
        pltpu.VMEM((16,), jnp.int32),
        pltpu.VMEM((16,), jnp.float32),
        pltpu.VMEM((16,), jnp.float32),
        pltpu.VMEM((256,), jnp.int32),
        pltpu.VMEM((256,), jnp.float32),
        pltpu.VMEM((_K,), jnp.float32),
    ],
)
def _ngram_sc(corpus_ref, lasts_ref, topk_ref, cmask_ref, out_ref, *scratch):
  _body(corpus_ref, lasts_ref, topk_ref, cmask_ref, out_ref, *scratch)


@jax.jit
def kernel(input_ids, topk_id, topk_prob, corpus, sw_mask, bd_mask):
  del topk_prob
  corpus_pad = jnp.concatenate([corpus, jnp.zeros((16,), jnp.int32)])
  lasts = jnp.concatenate([
      jnp.full((16,), input_ids[-2], jnp.int32),
      jnp.full((16,), input_ids[-1], jnp.int32),
  ])
  cmask = jnp.concatenate([
      bd_mask[topk_id].astype(jnp.float32),
      sw_mask[topk_id].astype(jnp.float32),
  ])
  return _ngram_sc(corpus_pad, lasts, topk_id, cmask)


# speedup vs baseline: 13.2388x; 1.1249x over previous
import functools

import jax
import jax.numpy as jnp
from jax import lax
from jax.experimental import pallas as pl
from jax.experimental.pallas import tpu as pltpu
from jax.experimental.pallas import tpu_sc as plsc

_N = 3
_BETA = 0.9
_SW_COEFF = 1.0
_VOCAB = 100000
_L = 32768
_K = 64

_NT = 16
_CHUNK = _L // _NT
_ZSLICE = 18752
_ZTOT = _NT * _ZSLICE
_TOT_OFF = _ZTOT
_ZSIZE = _ZTOT + _NT * 32
_PCHUNK = 6256
_PLAST = _VOCAB - 15 * _PCHUNK
_T1_SLOT = 300016
_T2_SLOT = 300017


def _body(corpus_ref, lasts_ref, topk_ref, cmask_ref, out_ref,
          ztab, cv, zbuf, lasts_v, topk_v, cmask_v,
          ones_s, val1_s, val2_s, idx0_s, idx1_s, idx2_s,
          val1_t, val2_t, idx0_t, idx1_t, idx2_t,
          tidx1, tidx2, tval1, tval2, gidx, gval, finalbuf):
  sid = lax.axis_index("s")
  zero16 = jnp.zeros((16,), jnp.float32)
  ones16 = jnp.ones((16,), jnp.float32)
  lanes = lax.iota(jnp.int32, 16)

  def _load_phase(s0, s1, s2, s3, s4, s5):
    d_cv = pltpu.async_copy(
        corpus_ref.at[pl.ds(sid * _CHUNK, _CHUNK + 16)], cv, s0)
    d_lasts = pltpu.async_copy(lasts_ref, lasts_v, s1)

    def _zfill(i, _):
      for c in range(8):
        zbuf[pl.ds(i * 128 + c * 16, 16)] = zero16
      return 0
    lax.fori_loop(0, _PCHUNK // 128, _zfill, 0)
    for c in range(_PCHUNK // 128 * 8, _PCHUNK // 16):
      zbuf[pl.ds(c * 16, 16)] = zero16
    for t in range(8):
      ones_s[pl.ds(t * 16, 16)] = ones16

    pofs = jnp.where(sid == _NT - 1, _VOCAB - _PCHUNK, sid * _PCHUNK)
    zb = sid * _ZSLICE
    rem = _ZSLICE - 2 * _PCHUNK
    d_z = [
        pltpu.async_copy(zbuf, out_ref.at[pl.ds(pofs, _PCHUNK)], s2),
        pltpu.async_copy(zbuf, ztab.at[pl.ds(zb, _PCHUNK)], s3),
        pltpu.async_copy(zbuf, ztab.at[pl.ds(zb + _PCHUNK, _PCHUNK)], s4),
        pltpu.async_copy(zbuf.at[pl.ds(0, rem)],
                         ztab.at[pl.ds(zb + 2 * _PCHUNK, rem)], s5),
    ]
    for d in d_z:
      d.wait()
    d_cv.wait()
    d_lasts.wait()

  pl.run_scoped(
      _load_phase,
      s0=pltpu.SemaphoreType.DMA(()), s1=pltpu.SemaphoreType.DMA(()),
      s2=pltpu.SemaphoreType.DMA(()), s3=pltpu.SemaphoreType.DMA(()),
      s4=pltpu.SemaphoreType.DMA(()), s5=pltpu.SemaphoreType.DMA(()))

  last2v = lasts_v[pl.ds(0, 16)]
  lastv = lasts_v[pl.ds(16, 16)]

  plsc.subcore_barrier()

  buf_a = (idx0_s, idx1_s, idx2_s, val1_s, val2_s)
  buf_b = (idx0_t, idx1_t, idx2_t, val1_t, val2_t)

  def _compute(g, bufs, t1acc, t2acc):
    i0, i1, i2, v1, v2 = bufs
    for t in range(8):
      p = g * 128 + t * 16
      cv0 = cv[pl.ds(p, 16)]
      cv1 = cv[pl.ds(p + 1, 16)]
      cv2 = cv[pl.ds(p + 2, 16)]
      posv = sid * _CHUNK + p + lanes
      m1 = (cv0 == lastv) & (posv <= _L - 2)
      m2 = (cv0 == last2v) & (cv1 == lastv) & (posv <= _L - 3)
      m1f = jnp.where(m1, 1.0, 0.0).astype(jnp.float32)
      m2f = jnp.where(m2, 1.0, 0.0).astype(jnp.float32)
      t1acc = t1acc + m1f
      t2acc = t2acc + m2f
      i0[pl.ds(t * 16, 16)] = cv0
      i1[pl.ds(t * 16, 16)] = cv1 + _VOCAB
      i2[pl.ds(t * 16, 16)] = cv2 + 2 * _VOCAB
      v1[pl.ds(t * 16, 16)] = m1f
      v2[pl.ds(t * 16, 16)] = m2f
    return t1acc, t2acc

  def _fire(bufs, sem):
    i0, i1, i2, v1, v2 = bufs
    return [
        pltpu.async_copy(ones_s, ztab.at[i0], sem, add=True),
        pltpu.async_copy(v1, ztab.at[i1], sem, add=True),
        pltpu.async_copy(v2, ztab.at[i2], sem, add=True),
    ]

  def _hist_phase(sem_a, sem_b):
    t1acc = zero16
    t2acc = zero16
    inflight = {0: None, 1: None}
    for g in range(_CHUNK // 128):
      par = g % 2
      bufs = buf_a if par == 0 else buf_b
      sem = sem_a if par == 0 else sem_b
      if inflight[par] is not None:
        for d in inflight[par]:
          d.wait()
      t1acc, t2acc = _compute(g, bufs, t1acc, t2acc)
      inflight[par] = _fire(bufs, sem)
    for par in (0, 1):
      if inflight[par] is not None:
        for d in inflight[par]:
          d.wait()
    tval1[pl.ds(0, 16)] = t1acc
    tval2[pl.ds(0, 16)] = t2acc

  pl.run_scoped(
      _hist_phase,
      sem_a=pltpu.SemaphoreType.DMA(()), sem_b=pltpu.SemaphoreType.DMA(()))

  tidx1[pl.ds(0, 16)] = jnp.full((16,), _T1_SLOT, jnp.int32)
  tidx2[pl.ds(0, 16)] = jnp.full((16,), _T2_SLOT, jnp.int32)
  pltpu.sync_copy(tval1, ztab.at[tidx1], add=True)
  pltpu.sync_copy(tval2, ztab.at[tidx2], add=True)

  plsc.subcore_barrier()

  @pl.when(sid == 0)
  def _():
    pltpu.sync_copy(topk_ref, topk_v)
    pltpu.sync_copy(cmask_ref, cmask_v)
    zero16i = jnp.zeros((16,), jnp.int32)
    for g in range(4):
      tk = topk_v[pl.ds(g * 16, 16)]
      gidx[pl.ds(g * 16, 16)] = tk
      gidx[pl.ds(64 + g * 16, 16)] = tk + _VOCAB
      gidx[pl.ds(128 + g * 16, 16)] = tk + 2 * _VOCAB
    gidx[pl.ds(192, 16)] = jnp.full((16,), _T1_SLOT, jnp.int32)
    gidx[pl.ds(208, 16)] = jnp.full((16,), _T2_SLOT, jnp.int32)
    gidx[pl.ds(224, 16)] = zero16i
    gidx[pl.ds(240, 16)] = zero16i
    pltpu.sync_copy(ztab.at[gidx.at[pl.ds(0, 128)]], gval.at[pl.ds(0, 128)])
    pltpu.sync_copy(ztab.at[gidx.at[pl.ds(128, 128)]],
                    gval.at[pl.ds(128, 128)])
    t1 = gval[pl.ds(192, 16)]
    t2 = gval[pl.ds(208, 16)]

    for g in range(4):
      c0 = gval[pl.ds(g * 16, 16)]
      c1 = gval[pl.ds(64 + g * 16, 16)]
      c2 = gval[pl.ds(128 + g * 16, 16)]
      bd = cmask_v[pl.ds(g * 16, 16)]
      sw = cmask_v[pl.ds(64 + g * 16, 16)]
      notbd = bd == 0.0
      remaining = ones16
      hit2 = (c2 > 0.0) & notbd
      score = jnp.where(hit2, _BETA * (c2 / (t2 + 1.0)), 0.0)
      remaining = jnp.where(hit2, remaining * (1.0 - _BETA), remaining)
      hit1 = (c1 > 0.0) & notbd
      score = score + jnp.where(hit1, remaining * _BETA * (c1 / (t1 + 1.0)),
                                0.0)
      remaining = jnp.where(hit1, remaining * (1.0 - _BETA), remaining)
      hit0 = (c0 > 0.0) & notbd
      score = score + jnp.where(hit0, remaining * (c0 / float(_L)), 0.0)
      fin = jnp.where(bd > 0.0, 0.0,
                      jnp.where(sw > 0.0, _SW_COEFF * score, score))
      finalbuf[pl.ds(g * 16, 16)] = fin

    pltpu.sync_copy(finalbuf, out_ref.at[topk_v])


@functools.partial(
    pl.kernel,
    out_type=jax.ShapeDtypeStruct((_VOCAB,), jnp.float32),
    mesh=plsc.VectorSubcoreMesh(
        core_axis_name="c", subcore_axis_name="s", num_cores=1),
    scratch_types=[
        pltpu.VMEM_SHARED((_ZSIZE,), jnp.float32),
        pltpu.VMEM((_CHUNK + 16,), jnp.int32),
        pltpu.VMEM((_PCHUNK,), jnp.float32),
        pltpu.VMEM((32,), jnp.int32),
        pltpu.VMEM((_K,), jnp.int32),
        pltpu.VMEM((2 * _K,), jnp.float32),
        pltpu.VMEM((128,), jnp.float32),
        pltpu.VMEM((128,), jnp.float32),
        pltpu.VMEM((128,), jnp.float32),
        pltpu.VMEM((128,), jnp.int32),
        pltpu.VMEM((128,), jnp.int32),
        pltpu.VMEM((128,), jnp.int32),
        pltpu.VMEM((128,), jnp.float32),
        pltpu.VMEM((128,), jnp.float32),
        pltpu.VMEM((128,), jnp.int32),
        pltpu.VMEM((128,), jnp.int32),
        pltpu.VMEM((128,), jnp.int32),
        pltpu.VMEM((16,), jnp.int32),
        pltpu.VMEM((16,), jnp.int32),
        pltpu.VMEM((16,), jnp.float32),
        pltpu.VMEM((16,), jnp.float32),
        pltpu.VMEM((256,), jnp.int32),
        pltpu.VMEM((256,), jnp.float32),
        pltpu.VMEM((_K,), jnp.float32),
    ],
)
def _ngram_sc(corpus_ref, lasts_ref, topk_ref, cmask_ref, out_ref, *scratch):
  _body(corpus_ref, lasts_ref, topk_ref, cmask_ref, out_ref, *scratch)


@jax.jit
def kernel(input_ids, topk_id, topk_prob, corpus, sw_mask, bd_mask):
  del topk_prob
  corpus_pad = jnp.concatenate([corpus, jnp.zeros((16,), jnp.int32)])
  lasts = jnp.concatenate([
      jnp.full((16,), input_ids[-2], jnp.int32),
      jnp.full((16,), input_ids[-1], jnp.int32),
  ])
  cmask = jnp.concatenate([
      bd_mask[topk_id].astype(jnp.float32),
      sw_mask[topk_id].astype(jnp.float32),
  ])
  return _ngram_sc(corpus_pad, lasts, topk_id, cmask)


# speedup vs baseline: 13.3867x; 1.0112x over previous
import functools

import jax
import jax.numpy as jnp
from jax import lax
from jax.experimental import pallas as pl
from jax.experimental.pallas import tpu as pltpu
from jax.experimental.pallas import tpu_sc as plsc

_N = 3
_BETA = 0.9
_SW_COEFF = 1.0
_VOCAB = 100000
_L = 32768
_K = 64

_NT = 16
_CHUNK = _L // _NT
_ZSLICE = 18752
_ZTOT = _NT * _ZSLICE
_TOT_OFF = _ZTOT
_ZSIZE = _ZTOT + _NT * 32
_PCHUNK = 6256
_PLAST = _VOCAB - 15 * _PCHUNK
_T1_SLOT = 300016
_T2_SLOT = 300017
_GRP = 256


def _body(corpus_ref, lasts_ref, topk_ref, cmask_ref, out_ref,
          ztab, cv, zbuf, lasts_v, topk_v, cmask_v,
          ones_s, val1_s, val2_s, idx0_s, idx1_s, idx2_s,
          val1_t, val2_t, idx0_t, idx1_t, idx2_t,
          tidx1, tidx2, tval1, tval2, gidx, gval, finalbuf):
  sid = lax.axis_index("s")
  zero16 = jnp.zeros((16,), jnp.float32)
  ones16 = jnp.ones((16,), jnp.float32)
  lanes = lax.iota(jnp.int32, 16)

  def _load_phase(s0, s1, s2, s3, s4, s5):
    d_cv = pltpu.async_copy(
        corpus_ref.at[pl.ds(sid * _CHUNK, _CHUNK + 16)], cv, s0)
    d_lasts = pltpu.async_copy(lasts_ref, lasts_v, s1)

    def _zfill(i, _):
      for c in range(8):
        zbuf[pl.ds(i * 128 + c * 16, 16)] = zero16
      return 0
    lax.fori_loop(0, _PCHUNK // 128, _zfill, 0)
    for c in range(_PCHUNK // 128 * 8, _PCHUNK // 16):
      zbuf[pl.ds(c * 16, 16)] = zero16
    for t in range(_GRP // 16):
      ones_s[pl.ds(t * 16, 16)] = ones16

    pofs = jnp.where(sid == _NT - 1, _VOCAB - _PCHUNK, sid * _PCHUNK)
    zb = sid * _ZSLICE
    rem = _ZSLICE - 2 * _PCHUNK
    d_z = [
        pltpu.async_copy(zbuf, out_ref.at[pl.ds(pofs, _PCHUNK)], s2),
        pltpu.async_copy(zbuf, ztab.at[pl.ds(zb, _PCHUNK)], s3),
        pltpu.async_copy(zbuf, ztab.at[pl.ds(zb + _PCHUNK, _PCHUNK)], s4),
        pltpu.async_copy(zbuf.at[pl.ds(0, rem)],
                         ztab.at[pl.ds(zb + 2 * _PCHUNK, rem)], s5),
    ]
    for d in d_z:
      d.wait()
    d_cv.wait()
    d_lasts.wait()

  pl.run_scoped(
      _load_phase,
      s0=pltpu.SemaphoreType.DMA(()), s1=pltpu.SemaphoreType.DMA(()),
      s2=pltpu.SemaphoreType.DMA(()), s3=pltpu.SemaphoreType.DMA(()),
      s4=pltpu.SemaphoreType.DMA(()), s5=pltpu.SemaphoreType.DMA(()))

  last2v = lasts_v[pl.ds(0, 16)]
  lastv = lasts_v[pl.ds(16, 16)]

  plsc.subcore_barrier()

  buf_a = (idx0_s, idx1_s, idx2_s, val1_s, val2_s)
  buf_b = (idx0_t, idx1_t, idx2_t, val1_t, val2_t)

  def _compute(g, bufs, t1acc, t2acc):
    i0, i1, i2, v1, v2 = bufs
    for t in range(_GRP // 16):
      p = g * _GRP + t * 16
      cv0 = cv[pl.ds(p, 16)]
      cv1 = cv[pl.ds(p + 1, 16)]
      cv2 = cv[pl.ds(p + 2, 16)]
      posv = sid * _CHUNK + p + lanes
      m1 = (cv0 == lastv) & (posv <= _L - 2)
      m2 = (cv0 == last2v) & (cv1 == lastv) & (posv <= _L - 3)
      m1f = jnp.where(m1, 1.0, 0.0).astype(jnp.float32)
      m2f = jnp.where(m2, 1.0, 0.0).astype(jnp.float32)
      t1acc = t1acc + m1f
      t2acc = t2acc + m2f
      i0[pl.ds(t * 16, 16)] = cv0
      i1[pl.ds(t * 16, 16)] = cv1 + _VOCAB
      i2[pl.ds(t * 16, 16)] = cv2 + 2 * _VOCAB
      v1[pl.ds(t * 16, 16)] = m1f
      v2[pl.ds(t * 16, 16)] = m2f
    return t1acc, t2acc

  def _fire(bufs, sem):
    i0, i1, i2, v1, v2 = bufs
    return [
        pltpu.async_copy(ones_s, ztab.at[i0], sem, add=True),
        pltpu.async_copy(v1, ztab.at[i1], sem, add=True),
        pltpu.async_copy(v2, ztab.at[i2], sem, add=True),
    ]

  def _hist_phase(sem_a, sem_b):
    t1acc = zero16
    t2acc = zero16
    inflight = {0: None, 1: None}
    for g in range(_CHUNK // _GRP):
      par = g % 2
      bufs = buf_a if par == 0 else buf_b
      sem = sem_a if par == 0 else sem_b
      if inflight[par] is not None:
        for d in inflight[par]:
          d.wait()
      t1acc, t2acc = _compute(g, bufs, t1acc, t2acc)
      inflight[par] = _fire(bufs, sem)
    for par in (0, 1):
      if inflight[par] is not None:
        for d in inflight[par]:
          d.wait()
    tval1[pl.ds(0, 16)] = t1acc
    tval2[pl.ds(0, 16)] = t2acc

  pl.run_scoped(
      _hist_phase,
      sem_a=pltpu.SemaphoreType.DMA(()), sem_b=pltpu.SemaphoreType.DMA(()))

  tidx1[pl.ds(0, 16)] = jnp.full((16,), _T1_SLOT, jnp.int32)
  tidx2[pl.ds(0, 16)] = jnp.full((16,), _T2_SLOT, jnp.int32)
  pltpu.sync_copy(tval1, ztab.at[tidx1], add=True)
  pltpu.sync_copy(tval2, ztab.at[tidx2], add=True)

  plsc.subcore_barrier()

  @pl.when(sid == 0)
  def _():
    pltpu.sync_copy(topk_ref, topk_v)
    pltpu.sync_copy(cmask_ref, cmask_v)
    zero16i = jnp.zeros((16,), jnp.int32)
    for g in range(4):
      tk = topk_v[pl.ds(g * 16, 16)]
      gidx[pl.ds(g * 16, 16)] = tk
      gidx[pl.ds(64 + g * 16, 16)] = tk + _VOCAB
      gidx[pl.ds(128 + g * 16, 16)] = tk + 2 * _VOCAB
    gidx[pl.ds(192, 16)] = jnp.full((16,), _T1_SLOT, jnp.int32)
    gidx[pl.ds(208, 16)] = jnp.full((16,), _T2_SLOT, jnp.int32)
    gidx[pl.ds(224, 16)] = zero16i
    gidx[pl.ds(240, 16)] = zero16i
    pltpu.sync_copy(ztab.at[gidx.at[pl.ds(0, 128)]], gval.at[pl.ds(0, 128)])
    pltpu.sync_copy(ztab.at[gidx.at[pl.ds(128, 128)]],
                    gval.at[pl.ds(128, 128)])
    t1 = gval[pl.ds(192, 16)]
    t2 = gval[pl.ds(208, 16)]

    for g in range(4):
      c0 = gval[pl.ds(g * 16, 16)]
      c1 = gval[pl.ds(64 + g * 16, 16)]
      c2 = gval[pl.ds(128 + g * 16, 16)]
      bd = cmask_v[pl.ds(g * 16, 16)]
      sw = cmask_v[pl.ds(64 + g * 16, 16)]
      notbd = bd == 0.0
      remaining = ones16
      hit2 = (c2 > 0.0) & notbd
      score = jnp.where(hit2, _BETA * (c2 / (t2 + 1.0)), 0.0)
      remaining = jnp.where(hit2, remaining * (1.0 - _BETA), remaining)
      hit1 = (c1 > 0.0) & notbd
      score = score + jnp.where(hit1, remaining * _BETA * (c1 / (t1 + 1.0)),
                                0.0)
      remaining = jnp.where(hit1, remaining * (1.0 - _BETA), remaining)
      hit0 = (c0 > 0.0) & notbd
      score = score + jnp.where(hit0, remaining * (c0 / float(_L)), 0.0)
      fin = jnp.where(bd > 0.0, 0.0,
                      jnp.where(sw > 0.0, _SW_COEFF * score, score))
      finalbuf[pl.ds(g * 16, 16)] = fin

    pltpu.sync_copy(finalbuf, out_ref.at[topk_v])


@functools.partial(
    pl.kernel,
    out_type=jax.ShapeDtypeStruct((_VOCAB,), jnp.float32),
    mesh=plsc.VectorSubcoreMesh(
        core_axis_name="c", subcore_axis_name="s", num_cores=1),
    scratch_types=[
        pltpu.VMEM_SHARED((_ZSIZE,), jnp.float32),
        pltpu.VMEM((_CHUNK + 16,), jnp.int32),
        pltpu.VMEM((_PCHUNK,), jnp.float32),
        pltpu.VMEM((32,), jnp.int32),
        pltpu.VMEM((_K,), jnp.int32),
        pltpu.VMEM((2 * _K,), jnp.float32),
        pltpu.VMEM((_GRP,), jnp.float32),
        pltpu.VMEM((_GRP,), jnp.float32),
        pltpu.VMEM((_GRP,), jnp.float32),
        pltpu.VMEM((_GRP,), jnp.int32),
        pltpu.VMEM((_GRP,), jnp.int32),
        pltpu.VMEM((_GRP,), jnp.int32),
        pltpu.VMEM((_GRP,), jnp.float32),
        pltpu.VMEM((_GRP,), jnp.float32),
        pltpu.VMEM((_GRP,), jnp.int32),
        pltpu.VMEM((_GRP,), jnp.int32),
        pltpu.VMEM((_GRP,), jnp.int32),
        pltpu.VMEM((16,), jnp.int32),
        pltpu.VMEM((16,), jnp.int32),
        pltpu.VMEM((16,), jnp.float32),
        pltpu.VMEM((16,), jnp.float32),
        pltpu.VMEM((256,), jnp.int32),
        pltpu.VMEM((256,), jnp.float32),
        pltpu.VMEM((_K,), jnp.float32),
    ],
)
def _ngram_sc(corpus_ref, lasts_ref, topk_ref, cmask_ref, out_ref, *scratch):
  _body(corpus_ref, lasts_ref, topk_ref, cmask_ref, out_ref, *scratch)


@jax.jit
def kernel(input_ids, topk_id, topk_prob, corpus, sw_mask, bd_mask):
  del topk_prob
  corpus_pad = jnp.concatenate([corpus, jnp.zeros((16,), jnp.int32)])
  lasts = jnp.concatenate([
      jnp.full((16,), input_ids[-2], jnp.int32),
      jnp.full((16,), input_ids[-1], jnp.int32),
  ])
  cmask = jnp.concatenate([
      bd_mask[topk_id].astype(jnp.float32),
      sw_mask[topk_id].astype(jnp.float32),
  ])
  return _ngram_sc(corpus_pad, lasts, topk_id, cmask)


# speedup vs baseline: 13.8487x; 1.0345x over previous
import functools

import jax
import jax.numpy as jnp
from jax import lax
from jax.experimental import pallas as pl
from jax.experimental.pallas import tpu as pltpu
from jax.experimental.pallas import tpu_sc as plsc

_N = 3
_BETA = 0.9
_SW_COEFF = 1.0
_VOCAB = 100000
_L = 32768
_K = 64

_NT = 16
_CHUNK = _L // _NT
_ZSLICE = 18752
_ZTOT = _NT * _ZSLICE
_TOT_OFF = _ZTOT
_ZSIZE = _ZTOT + _NT * 32
_PCHUNK = 6256
_PLAST = _VOCAB - 15 * _PCHUNK
_T1_SLOT = 300016
_T2_SLOT = 300017
_GRP = 256


def _body(corpus_ref, lasts_ref, topk_ref, cmask_ref, out_ref,
          ztab, cv, zbuf, lasts_v, topk_v, cmask_v,
          ones_s, val1_s, val2_s, idx0_s, idx1_s, idx2_s,
          val1_t, val2_t, idx0_t, idx1_t, idx2_t,
          tidx1, tval1, gidx, gval, finalbuf):
  sid = lax.axis_index("s")
  zero16 = jnp.zeros((16,), jnp.float32)
  ones16 = jnp.ones((16,), jnp.float32)
  lanes = lax.iota(jnp.int32, 16)

  def _load_phase(s0, s1, s2, s3, s4, s5):
    d_cv = pltpu.async_copy(
        corpus_ref.at[pl.ds(sid * _CHUNK, _CHUNK + 16)], cv, s0)
    d_lasts = pltpu.async_copy(lasts_ref, lasts_v, s1)

    @pl.when(sid == 0)
    def _():
      d_tk = pltpu.async_copy(topk_ref, topk_v, s2)
      d_cm = pltpu.async_copy(cmask_ref, cmask_v, s3)
      d_tk.wait()
      d_cm.wait()
      for g in range(4):
        tk = topk_v[pl.ds(g * 16, 16)]
        gidx[pl.ds(g * 16, 16)] = tk
        gidx[pl.ds(64 + g * 16, 16)] = tk + _VOCAB
        gidx[pl.ds(128 + g * 16, 16)] = tk + 2 * _VOCAB
      gidx[pl.ds(192, 16)] = jnp.full((16,), _T1_SLOT, jnp.int32)
      gidx[pl.ds(208, 16)] = jnp.full((16,), _T2_SLOT, jnp.int32)
      zero16i = jnp.zeros((16,), jnp.int32)
      gidx[pl.ds(224, 16)] = zero16i
      gidx[pl.ds(240, 16)] = zero16i

    def _zfill(i, _):
      for c in range(8):
        zbuf[pl.ds(i * 128 + c * 16, 16)] = zero16
      return 0
    lax.fori_loop(0, _PCHUNK // 128, _zfill, 0)
    for c in range(_PCHUNK // 128 * 8, _PCHUNK // 16):
      zbuf[pl.ds(c * 16, 16)] = zero16
    for t in range(_GRP // 16):
      ones_s[pl.ds(t * 16, 16)] = ones16

    pofs = jnp.where(sid == _NT - 1, _VOCAB - _PCHUNK, sid * _PCHUNK)
    zb = sid * _ZSLICE
    rem = _ZSLICE - 2 * _PCHUNK
    d_z = [
        pltpu.async_copy(zbuf, out_ref.at[pl.ds(pofs, _PCHUNK)], s2),
        pltpu.async_copy(zbuf, ztab.at[pl.ds(zb, _PCHUNK)], s3),
        pltpu.async_copy(zbuf, ztab.at[pl.ds(zb + _PCHUNK, _PCHUNK)], s4),
        pltpu.async_copy(zbuf.at[pl.ds(0, rem)],
                         ztab.at[pl.ds(zb + 2 * _PCHUNK, rem)], s5),
    ]
    for d in d_z:
      d.wait()
    d_cv.wait()
    d_lasts.wait()

  pl.run_scoped(
      _load_phase,
      s0=pltpu.SemaphoreType.DMA(()), s1=pltpu.SemaphoreType.DMA(()),
      s2=pltpu.SemaphoreType.DMA(()), s3=pltpu.SemaphoreType.DMA(()),
      s4=pltpu.SemaphoreType.DMA(()), s5=pltpu.SemaphoreType.DMA(()))

  last2v = lasts_v[pl.ds(0, 16)]
  lastv = lasts_v[pl.ds(16, 16)]

  plsc.subcore_barrier()

  buf_a = (idx0_s, idx1_s, idx2_s, val1_s, val2_s)
  buf_b = (idx0_t, idx1_t, idx2_t, val1_t, val2_t)

  def _compute(g, bufs, t1acc, t2acc):
    i0, i1, i2, v1, v2 = bufs
    for t in range(_GRP // 16):
      p = g * _GRP + t * 16
      cv0 = cv[pl.ds(p, 16)]
      cv1 = cv[pl.ds(p + 1, 16)]
      cv2 = cv[pl.ds(p + 2, 16)]
      posv = sid * _CHUNK + p + lanes
      m1 = (cv0 == lastv) & (posv <= _L - 2)
      m2 = (cv0 == last2v) & (cv1 == lastv) & (posv <= _L - 3)
      m1f = jnp.where(m1, 1.0, 0.0).astype(jnp.float32)
      m2f = jnp.where(m2, 1.0, 0.0).astype(jnp.float32)
      t1acc = t1acc + m1f
      t2acc = t2acc + m2f
      i0[pl.ds(t * 16, 16)] = cv0
      i1[pl.ds(t * 16, 16)] = cv1 + _VOCAB
      i2[pl.ds(t * 16, 16)] = cv2 + 2 * _VOCAB
      v1[pl.ds(t * 16, 16)] = m1f
      v2[pl.ds(t * 16, 16)] = m2f
    return t1acc, t2acc

  def _fire(bufs, sem):
    i0, i1, i2, v1, v2 = bufs
    return [
        pltpu.async_copy(ones_s, ztab.at[i0], sem, add=True),
        pltpu.async_copy(v1, ztab.at[i1], sem, add=True),
        pltpu.async_copy(v2, ztab.at[i2], sem, add=True),
    ]

  def _hist_phase(sem_a, sem_b):
    t1acc = zero16
    t2acc = zero16
    inflight = {0: None, 1: None}
    for g in range(_CHUNK // _GRP):
      par = g % 2
      bufs = buf_a if par == 0 else buf_b
      sem = sem_a if par == 0 else sem_b
      if inflight[par] is not None:
        for d in inflight[par]:
          d.wait()
      t1acc, t2acc = _compute(g, bufs, t1acc, t2acc)
      inflight[par] = _fire(bufs, sem)
    for par in (0, 1):
      if inflight[par] is not None:
        for d in inflight[par]:
          d.wait()
    tval1[pl.ds(0, 16)] = t1acc
    tval1[pl.ds(16, 16)] = t2acc

  pl.run_scoped(
      _hist_phase,
      sem_a=pltpu.SemaphoreType.DMA(()), sem_b=pltpu.SemaphoreType.DMA(()))

  tidx1[pl.ds(0, 16)] = jnp.full((16,), _T1_SLOT, jnp.int32)
  tidx1[pl.ds(16, 16)] = jnp.full((16,), _T2_SLOT, jnp.int32)
  pltpu.sync_copy(tval1, ztab.at[tidx1], add=True)

  plsc.subcore_barrier()

  @pl.when(sid == 0)
  def _():
    def _gather_counts(sg0, sg1):
      d0 = pltpu.async_copy(
          ztab.at[gidx.at[pl.ds(0, 128)]], gval.at[pl.ds(0, 128)], sg0)
      d1 = pltpu.async_copy(
          ztab.at[gidx.at[pl.ds(128, 128)]], gval.at[pl.ds(128, 128)], sg1)
      d0.wait()
      d1.wait()

    pl.run_scoped(_gather_counts, sg0=pltpu.SemaphoreType.DMA(()),
                  sg1=pltpu.SemaphoreType.DMA(()))
    t1 = gval[pl.ds(192, 16)]
    t2 = gval[pl.ds(208, 16)]

    for g in range(4):
      c0 = gval[pl.ds(g * 16, 16)]
      c1 = gval[pl.ds(64 + g * 16, 16)]
      c2 = gval[pl.ds(128 + g * 16, 16)]
      bd = cmask_v[pl.ds(g * 16, 16)]
      sw = cmask_v[pl.ds(64 + g * 16, 16)]
      notbd = bd == 0.0
      remaining = ones16
      hit2 = (c2 > 0.0) & notbd
      score = jnp.where(hit2, _BETA * (c2 / (t2 + 1.0)), 0.0)
      remaining = jnp.where(hit2, remaining * (1.0 - _BETA), remaining)
      hit1 = (c1 > 0.0) & notbd
      score = score + jnp.where(hit1, remaining * _BETA * (c1 / (t1 + 1.0)),
                                0.0)
      remaining = jnp.where(hit1, remaining * (1.0 - _BETA), remaining)
      hit0 = (c0 > 0.0) & notbd
      score = score + jnp.where(hit0, remaining * (c0 / float(_L)), 0.0)
      fin = jnp.where(bd > 0.0, 0.0,
                      jnp.where(sw > 0.0, _SW_COEFF * score, score))
      finalbuf[pl.ds(g * 16, 16)] = fin

    pltpu.sync_copy(finalbuf, out_ref.at[topk_v])


@functools.partial(
    pl.kernel,
    out_type=jax.ShapeDtypeStruct((_VOCAB,), jnp.float32),
    mesh=plsc.VectorSubcoreMesh(
        core_axis_name="c", subcore_axis_name="s", num_cores=1),
    scratch_types=[
        pltpu.VMEM_SHARED((_ZSIZE,), jnp.float32),
        pltpu.VMEM((_CHUNK + 16,), jnp.int32),
        pltpu.VMEM((_PCHUNK,), jnp.float32),
        pltpu.VMEM((32,), jnp.int32),
        pltpu.VMEM((_K,), jnp.int32),
        pltpu.VMEM((2 * _K,), jnp.float32),
        pltpu.VMEM((_GRP,), jnp.float32),
        pltpu.VMEM((_GRP,), jnp.float32),
        pltpu.VMEM((_GRP,), jnp.float32),
        pltpu.VMEM((_GRP,), jnp.int32),
        pltpu.VMEM((_GRP,), jnp.int32),
        pltpu.VMEM((_GRP,), jnp.int32),
        pltpu.VMEM((_GRP,), jnp.float32),
        pltpu.VMEM((_GRP,), jnp.float32),
        pltpu.VMEM((_GRP,), jnp.int32),
        pltpu.VMEM((_GRP,), jnp.int32),
        pltpu.VMEM((_GRP,), jnp.int32),
        pltpu.VMEM((32,), jnp.int32),
        pltpu.VMEM((32,), jnp.float32),
        pltpu.VMEM((256,), jnp.int32),
        pltpu.VMEM((256,), jnp.float32),
        pltpu.VMEM((_K,), jnp.float32),
    ],
)
def _ngram_sc(corpus_ref, lasts_ref, topk_ref, cmask_ref, out_ref, *scratch):
  _body(corpus_ref, lasts_ref, topk_ref, cmask_ref, out_ref, *scratch)


@jax.jit
def kernel(input_ids, topk_id, topk_prob, corpus, sw_mask, bd_mask):
  del topk_prob
  corpus_pad = jnp.concatenate([corpus, jnp.zeros((16,), jnp.int32)])
  lasts = jnp.concatenate([
      jnp.full((16,), input_ids[-2], jnp.int32),
      jnp.full((16,), input_ids[-1], jnp.int32),
  ])
  cmask = jnp.concatenate([
      bd_mask[topk_id].astype(jnp.float32),
      sw_mask[topk_id].astype(jnp.float32),
  ])
  return _ngram_sc(corpus_pad, lasts, topk_id, cmask)
